# baseline (speedup 1.0000x reference)
import jax
import jax.numpy as jnp
from jax.experimental import pallas as pl
from jax.experimental.pallas import tpu as pltpu
from jax.experimental.pallas import tpu_sc as plsc

_RB = 8
_L = 16


def kernel(x, pe_table):
    B, S, H = x.shape
    x2 = x.reshape(B * S, H)
    n_pe_blocks = S // _RB

    mesh = plsc.VectorSubcoreMesh(core_axis_name="c", subcore_axis_name="s")

    @pl.kernel(out_type=jax.ShapeDtypeStruct((B * S, H), x.dtype), mesh=mesh)
    def pe_add_sc(x_hbm, pe_hbm, o_hbm):
        def body(x_vmem, pe_vmem, o_vmem):
            @pl.loop(0, _RB)
            def _row(r):
                @pl.loop(0, H, step=_L)
                def _col(c):
                    slc = (pl.ds(r, 1), pl.ds(c, _L))
                    o_vmem.at[slc][...] = (
                        x_vmem.at[slc][...] + pe_vmem.at[slc][...]
                    )

        pltpu.emit_pipeline(
            body,
            grid=(B * S // _RB,),
            in_specs=[
                pl.BlockSpec((_RB, H), lambda i: (i, 0)),
                pl.BlockSpec((_RB, H), lambda i: (i % n_pe_blocks, 0)),
            ],
            out_specs=[pl.BlockSpec((_RB, H), lambda i: (i, 0))],
            core_axis_name=("c", "s"),
            dimension_semantics=(pltpu.PARALLEL,),
        )(x_hbm, pe_hbm, o_hbm)

    return pe_add_sc(x2, pe_table).reshape(B, S, H)
